# baseline (speedup 1.0000x reference)
import functools

import jax
import jax.numpy as jnp
from jax import lax
from jax.experimental import pallas as pl
from jax.experimental.pallas import tpu as pltpu
from jax.experimental.pallas import tpu_sc as plsc

NUM_CORES = 1
NUM_SUBCORES = 16
NW = NUM_CORES * NUM_SUBCORES

BPC = 2
NBUF = 2


def _emb_kernel(n_chunks, seq, d, idx_hbm, table_hbm, out_hbm,
                idx_v, rows_v, sems_g, sems_w):
    wid = lax.axis_index("s") * NUM_CORES + lax.axis_index("c")
    batch_base = wid * (n_chunks * BPC)
    n_groups = n_chunks // NBUF

    pltpu.sync_copy(idx_hbm.at[wid], idx_v)

    def gather_desc(i, s):
        return pltpu.make_async_copy(table_hbm.at[idx_v.at[i]], rows_v[s],
                                     sems_g[s])

    def write_descs(i, s):
        b0 = batch_base + i * BPC
        return [
            pltpu.make_async_copy(rows_v[s].at[pl.ds(k * seq, seq)],
                                  out_hbm.at[b0 + k], sems_w[s])
            for k in range(BPC)
        ]

    for s in range(NBUF):
        gather_desc(s, s).start()

    def body(j, _):
        for s in range(NBUF):
            i = j * NBUF + s
            gather_desc(i - NBUF, s).wait()
            for wd in write_descs(i - NBUF, s):
                wd.start()
            for wd in write_descs(i - NBUF, s):
                wd.wait()
            gather_desc(i, s).start()
        return _

    lax.fori_loop(1, n_groups, body, None)

    for s in range(NBUF):
        i = (n_groups - 1) * NBUF + s
        gather_desc(i, s).wait()
        for wd in write_descs(i, s):
            wd.start()
    for s in range(NBUF):
        i = (n_groups - 1) * NBUF + s
        for wd in write_descs(i, s):
            wd.wait()


def kernel(token_ids, weight):
    batch, seq = token_ids.shape
    n, d = weight.shape
    b = batch * seq
    assert batch % (NW * BPC * NBUF) == 0
    n_chunks = batch // (NW * BPC)

    flat_ids = token_ids.reshape(NW, n_chunks, BPC * seq).astype(jnp.int32)

    mesh = plsc.VectorSubcoreMesh(
        core_axis_name="c", subcore_axis_name="s",
        num_cores=NUM_CORES, num_subcores=NUM_SUBCORES)

    run = pl.kernel(
        functools.partial(_emb_kernel, n_chunks, seq, d),
        out_type=jax.ShapeDtypeStruct((batch, seq, d), jnp.float32),
        mesh=mesh,
        scratch_types=[
            pltpu.VMEM((n_chunks, BPC * seq), jnp.int32),
            [pltpu.VMEM((BPC * seq, d), jnp.float32) for _ in range(NBUF)],
            [pltpu.SemaphoreType.DMA for _ in range(NBUF)],
            [pltpu.SemaphoreType.DMA for _ in range(NBUF)],
        ],
        compiler_params=pltpu.CompilerParams(use_tc_tiling_on_sc=False),
    )
    return run(flat_ids, weight)


# speedup vs baseline: 1.0403x; 1.0403x over previous
import functools

import jax
import jax.numpy as jnp
from jax import lax
from jax.experimental import pallas as pl
from jax.experimental.pallas import tpu as pltpu
from jax.experimental.pallas import tpu_sc as plsc

NUM_CORES = 2
NUM_SUBCORES = 16
NW = NUM_CORES * NUM_SUBCORES

BPC = 2
NBUF = 2


def _emb_kernel(n_chunks, seq, d, idx_hbm, table_hbm, out_hbm,
                idx_v, rows_v, sems_g, sems_w):
    wid = lax.axis_index("s") * NUM_CORES + lax.axis_index("c")
    batch_base = wid * (n_chunks * BPC)
    n_groups = n_chunks // NBUF

    pltpu.sync_copy(idx_hbm.at[wid], idx_v)

    def gather_desc(i, s):
        return pltpu.make_async_copy(table_hbm.at[idx_v.at[i]], rows_v[s],
                                     sems_g[s])

    def write_descs(i, s):
        b0 = batch_base + i * BPC
        return [
            pltpu.make_async_copy(rows_v[s].at[pl.ds(k * seq, seq)],
                                  out_hbm.at[b0 + k], sems_w[s])
            for k in range(BPC)
        ]

    for s in range(NBUF):
        gather_desc(s, s).start()

    def body(j, _):
        for s in range(NBUF):
            i = j * NBUF + s
            gather_desc(i - NBUF, s).wait()
            for wd in write_descs(i - NBUF, s):
                wd.start()
            for wd in write_descs(i - NBUF, s):
                wd.wait()
            gather_desc(i, s).start()
        return _

    lax.fori_loop(1, n_groups, body, None)

    for s in range(NBUF):
        i = (n_groups - 1) * NBUF + s
        gather_desc(i, s).wait()
        for wd in write_descs(i, s):
            wd.start()
    for s in range(NBUF):
        i = (n_groups - 1) * NBUF + s
        for wd in write_descs(i, s):
            wd.wait()


def kernel(token_ids, weight):
    batch, seq = token_ids.shape
    n, d = weight.shape
    b = batch * seq
    assert batch % (NW * BPC * NBUF) == 0
    n_chunks = batch // (NW * BPC)

    flat_ids = token_ids.reshape(NW, n_chunks, BPC * seq).astype(jnp.int32)

    mesh = plsc.VectorSubcoreMesh(
        core_axis_name="c", subcore_axis_name="s",
        num_cores=NUM_CORES, num_subcores=NUM_SUBCORES)

    run = pl.kernel(
        functools.partial(_emb_kernel, n_chunks, seq, d),
        out_type=jax.ShapeDtypeStruct((batch, seq, d), jnp.float32),
        mesh=mesh,
        scratch_types=[
            pltpu.VMEM((n_chunks, BPC * seq), jnp.int32),
            [pltpu.VMEM((BPC * seq, d), jnp.float32) for _ in range(NBUF)],
            [pltpu.SemaphoreType.DMA for _ in range(NBUF)],
            [pltpu.SemaphoreType.DMA for _ in range(NBUF)],
        ],
        compiler_params=pltpu.CompilerParams(use_tc_tiling_on_sc=False),
    )
    return run(flat_ids, weight)
